# Initial kernel scaffold; baseline (speedup 1.0000x reference)
import jax
import jax.numpy as jnp
from jax.experimental import pallas as pl


def kernel(h, u, v):
    raise NotImplementedError("write your pallas kernel here")


# speedup vs baseline: 1.1026x; 1.1026x over previous
import jax
import jax.numpy as jnp
from jax import lax
from jax.experimental import pallas as pl
from jax.experimental.pallas import tpu as pltpu, tpu_sc as plsc

N_NODES = 10000
N_EDGES = 320000
D_FEAT = 128

NUM_CORES = 2
NUM_SUBCORES = 16
NUM_WORKERS = NUM_CORES * NUM_SUBCORES
EDGES_PER_WORKER = N_EDGES // NUM_WORKERS
CHUNK = 80
NUM_CHUNKS = EDGES_PER_WORKER // CHUNK
LANES = 16
FEAT_SLICES = D_FEAT // LANES


def _sc_body(h_hbm, u_hbm, v_hbm, out_hbm, idx_u, idx_v, rows_u, rows_v,
             out_v, sem_u, sem_v):
  wid = lax.axis_index("s") * NUM_CORES + lax.axis_index("c")
  base_w = pl.multiple_of(wid * EDGES_PER_WORKER, 8)

  def chunk_body(ci, _):
    base = pl.multiple_of(base_w + ci * CHUNK, 8)
    pltpu.sync_copy(u_hbm.at[pl.ds(base, CHUNK)], idx_u)
    pltpu.sync_copy(v_hbm.at[pl.ds(base, CHUNK)], idx_v)
    cp_u = pltpu.async_copy(h_hbm.at[idx_u], rows_u, sem_u)
    cp_v = pltpu.async_copy(h_hbm.at[idx_v], rows_v, sem_v)
    cp_u.wait()
    cp_v.wait()

    lane_iota = lax.iota(jnp.int32, LANES)

    def group_body(g, _):
      e0 = g * LANES
      row_idx = e0 + lane_iota
      acc = jnp.zeros((LANES,), jnp.float32)
      for f in range(D_FEAT):
        col = jnp.full((LANES,), f, jnp.int32)
        hu = plsc.load_gather(rows_u, [row_idx, col])
        hv = plsc.load_gather(rows_v, [row_idx, col])
        acc = acc + hu * hv
      out_v[pl.ds(e0, LANES)] = acc
      return 0

    lax.fori_loop(0, CHUNK // LANES, group_body, 0)
    pltpu.sync_copy(out_v, out_hbm.at[pl.ds(base, CHUNK)])
    return 0

  lax.fori_loop(0, NUM_CHUNKS, chunk_body, 0)


@jax.jit
def kernel(h, u, v):
  mesh = plsc.VectorSubcoreMesh(core_axis_name="c", subcore_axis_name="s")
  return pl.kernel(
      _sc_body,
      out_type=jax.ShapeDtypeStruct((N_EDGES,), jnp.float32),
      mesh=mesh,
      compiler_params=pltpu.CompilerParams(needs_layout_passes=False),
      scratch_types=[
          pltpu.VMEM((CHUNK,), jnp.int32),
          pltpu.VMEM((CHUNK,), jnp.int32),
          pltpu.VMEM((CHUNK, D_FEAT), jnp.float32),
          pltpu.VMEM((CHUNK, D_FEAT), jnp.float32),
          pltpu.VMEM((CHUNK,), jnp.float32),
          pltpu.SemaphoreType.DMA,
          pltpu.SemaphoreType.DMA,
      ],
  )(h, u, v)


# speedup vs baseline: 1.3361x; 1.2117x over previous
import jax
import jax.numpy as jnp
from jax import lax
from jax.experimental import pallas as pl
from jax.experimental.pallas import tpu as pltpu, tpu_sc as plsc

N_NODES = 10000
N_EDGES = 320000
D_FEAT = 128

NUM_CORES = 2
NUM_SUBCORES = 16
NUM_WORKERS = NUM_CORES * NUM_SUBCORES
EDGES_PER_WORKER = N_EDGES // NUM_WORKERS
CHUNK = 80
NUM_CHUNKS = EDGES_PER_WORKER // CHUNK
NBUF = 5
GROUPS = NUM_CHUNKS // NBUF
LANES = 16


def _sc_body(h_hbm, u_hbm, v_hbm, out_hbm, idx_u, idx_v, rows_u, rows_v,
             out_v, gsem_u, gsem_v, osem):
  wid = lax.axis_index("s") * NUM_CORES + lax.axis_index("c")
  base_w = pl.multiple_of(wid * EDGES_PER_WORKER, 8)
  lane_iota = lax.iota(jnp.int32, LANES)

  pltpu.sync_copy(u_hbm.at[pl.ds(base_w, EDGES_PER_WORKER)], idx_u)
  pltpu.sync_copy(v_hbm.at[pl.ds(base_w, EDGES_PER_WORKER)], idx_v)

  def issue_gathers(ci, b):
    off = pl.multiple_of(ci * CHUNK, 8)
    pltpu.async_copy(h_hbm.at[idx_u.at[pl.ds(off, CHUNK)]], rows_u.at[b],
                     gsem_u.at[b])
    pltpu.async_copy(h_hbm.at[idx_v.at[pl.ds(off, CHUNK)]], rows_v.at[b],
                     gsem_v.at[b])

  def wait_gathers(ci, b):
    off = pl.multiple_of(ci * CHUNK, 8)
    pltpu.make_async_copy(h_hbm.at[idx_u.at[pl.ds(off, CHUNK)]],
                          rows_u.at[b], gsem_u.at[b]).wait()
    pltpu.make_async_copy(h_hbm.at[idx_v.at[pl.ds(off, CHUNK)]],
                          rows_v.at[b], gsem_v.at[b]).wait()

  def out_slice(ci):
    return out_hbm.at[pl.ds(pl.multiple_of(base_w + ci * CHUNK, 8), CHUNK)]

  def compute_chunk(b):
    bvec = jnp.full((LANES,), b, jnp.int32)

    def group_body(gg, _):
      e0 = gg * LANES
      row_idx = e0 + lane_iota
      acc = jnp.zeros((LANES,), jnp.float32)
      for f in range(D_FEAT):
        col = jnp.full((LANES,), f, jnp.int32)
        hu = plsc.load_gather(rows_u, [bvec, row_idx, col])
        hv = plsc.load_gather(rows_v, [bvec, row_idx, col])
        acc = acc + hu * hv
      out_v[b, pl.ds(e0, LANES)] = acc
      return 0

    lax.fori_loop(0, CHUNK // LANES, group_body, 0)

  for b in range(NBUF):
    issue_gathers(b, b)

  def group(g, _):
    for b in range(NBUF):
      ci = g * NBUF + b
      wait_gathers(ci, b)

      @pl.when(g > 0)
      def _wait_prev_out():
        pltpu.make_async_copy(out_v.at[b], out_slice(ci - NBUF),
                              osem.at[b]).wait()

      compute_chunk(b)
      pltpu.async_copy(out_v.at[b], out_slice(ci), osem.at[b])

      @pl.when(g < GROUPS - 1)
      def _issue_next():
        issue_gathers(ci + NBUF, b)

    return 0

  lax.fori_loop(0, GROUPS, group, 0)

  for b in range(NBUF):
    ci = (GROUPS - 1) * NBUF + b
    pltpu.make_async_copy(out_v.at[b], out_slice(ci), osem.at[b]).wait()


@jax.jit
def kernel(h, u, v):
  mesh = plsc.VectorSubcoreMesh(core_axis_name="c", subcore_axis_name="s")
  return pl.kernel(
      _sc_body,
      out_type=jax.ShapeDtypeStruct((N_EDGES,), jnp.float32),
      mesh=mesh,
      compiler_params=pltpu.CompilerParams(needs_layout_passes=False),
      scratch_types=[
          pltpu.VMEM((EDGES_PER_WORKER,), jnp.int32),
          pltpu.VMEM((EDGES_PER_WORKER,), jnp.int32),
          pltpu.VMEM((NBUF, CHUNK, D_FEAT), jnp.float32),
          pltpu.VMEM((NBUF, CHUNK, D_FEAT), jnp.float32),
          pltpu.VMEM((NBUF, CHUNK), jnp.float32),
          pltpu.SemaphoreType.DMA((NBUF,)),
          pltpu.SemaphoreType.DMA((NBUF,)),
          pltpu.SemaphoreType.DMA((NBUF,)),
      ],
  )(h, u, v)


# speedup vs baseline: 11.3700x; 8.5102x over previous
import jax
import jax.numpy as jnp
from jax import lax
from jax.experimental import pallas as pl
from jax.experimental.pallas import tpu as pltpu, tpu_sc as plsc

N_NODES = 10000
N_EDGES = 320000
D_FEAT = 128

NUM_CORES = 2
NUM_SUBCORES = 16
NUM_WORKERS = NUM_CORES * NUM_SUBCORES
EDGES_PER_WORKER = N_EDGES // NUM_WORKERS
CHUNK = 80
NUM_CHUNKS = EDGES_PER_WORKER // CHUNK
NBUF = 5
GROUPS = NUM_CHUNKS // NBUF
LANES = 16


def _sc_body(h_hbm, u_hbm, v_hbm, out_hbm, idx_u, idx_v, rows_u, rows_v,
             out_v, gsem_u, gsem_v, osem):
  wid = lax.axis_index("s") * NUM_CORES + lax.axis_index("c")
  base_w = pl.multiple_of(wid * EDGES_PER_WORKER, 8)
  lane_iota = lax.iota(jnp.int32, LANES)

  pltpu.sync_copy(u_hbm.at[pl.ds(base_w, EDGES_PER_WORKER)], idx_u)
  pltpu.sync_copy(v_hbm.at[pl.ds(base_w, EDGES_PER_WORKER)], idx_v)

  def issue_gathers(ci, b):
    off = pl.multiple_of(ci * CHUNK, 8)
    pltpu.async_copy(h_hbm.at[idx_u.at[pl.ds(off, CHUNK)]], rows_u.at[b],
                     gsem_u.at[b])
    pltpu.async_copy(h_hbm.at[idx_v.at[pl.ds(off, CHUNK)]], rows_v.at[b],
                     gsem_v.at[b])

  def wait_gathers(ci, b):
    off = pl.multiple_of(ci * CHUNK, 8)
    pltpu.make_async_copy(h_hbm.at[idx_u.at[pl.ds(off, CHUNK)]],
                          rows_u.at[b], gsem_u.at[b]).wait()
    pltpu.make_async_copy(h_hbm.at[idx_v.at[pl.ds(off, CHUNK)]],
                          rows_v.at[b], gsem_v.at[b]).wait()

  def out_slice(ci):
    return out_hbm.at[pl.ds(pl.multiple_of(base_w + ci * CHUNK, 8), CHUNK)]

  def compute_chunk(b):
    def group_body(gg, _):
      e0 = gg * LANES

      def edge_body(k, score):
        e = e0 + k
        acc = rows_u[b, e, pl.ds(0, LANES)] * rows_v[b, e, pl.ds(0, LANES)]
        for j in range(1, D_FEAT // LANES):
          acc += (rows_u[b, e, pl.ds(j * LANES, LANES)] *
                  rows_v[b, e, pl.ds(j * LANES, LANES)])
        return jnp.where(lane_iota == k, jnp.sum(acc), score)

      score = lax.fori_loop(0, LANES, edge_body,
                            jnp.zeros((LANES,), jnp.float32), unroll=4)
      out_v[b, pl.ds(e0, LANES)] = score
      return 0

    lax.fori_loop(0, CHUNK // LANES, group_body, 0)

  for b in range(NBUF):
    issue_gathers(b, b)

  def group(g, _):
    for b in range(NBUF):
      ci = g * NBUF + b
      wait_gathers(ci, b)

      @pl.when(g > 0)
      def _wait_prev_out():
        pltpu.make_async_copy(out_v.at[b], out_slice(ci - NBUF),
                              osem.at[b]).wait()

      compute_chunk(b)
      pltpu.async_copy(out_v.at[b], out_slice(ci), osem.at[b])

      @pl.when(g < GROUPS - 1)
      def _issue_next():
        issue_gathers(ci + NBUF, b)

    return 0

  lax.fori_loop(0, GROUPS, group, 0)

  for b in range(NBUF):
    ci = (GROUPS - 1) * NBUF + b
    pltpu.make_async_copy(out_v.at[b], out_slice(ci), osem.at[b]).wait()


@jax.jit
def kernel(h, u, v):
  mesh = plsc.VectorSubcoreMesh(core_axis_name="c", subcore_axis_name="s")
  return pl.kernel(
      _sc_body,
      out_type=jax.ShapeDtypeStruct((N_EDGES,), jnp.float32),
      mesh=mesh,
      compiler_params=pltpu.CompilerParams(needs_layout_passes=False),
      scratch_types=[
          pltpu.VMEM((EDGES_PER_WORKER,), jnp.int32),
          pltpu.VMEM((EDGES_PER_WORKER,), jnp.int32),
          pltpu.VMEM((NBUF, CHUNK, D_FEAT), jnp.float32),
          pltpu.VMEM((NBUF, CHUNK, D_FEAT), jnp.float32),
          pltpu.VMEM((NBUF, CHUNK), jnp.float32),
          pltpu.SemaphoreType.DMA((NBUF,)),
          pltpu.SemaphoreType.DMA((NBUF,)),
          pltpu.SemaphoreType.DMA((NBUF,)),
      ],
  )(h, u, v)
